# baseline (speedup 1.0000x reference)
import functools

import numpy as np
import jax
import jax.numpy as jnp
from jax import lax
from jax.experimental import pallas as pl
from jax.experimental.pallas import tpu as pltpu
from jax.experimental.pallas import tpu_sc as plsc

D = 256
H = 8
L = 4
P = 4
C = 32
B = 4
Q = 2048
SPATIAL = np.array([[64, 64], [32, 32], [16, 16], [8, 8]], dtype=np.int64)
LEN_V = int(SPATIAL.prod(1).sum())
NCOL = H * L * P * 4

QB = 256
NW = 32
CHB = 16
SS = 8
NBAGS = B * Q * H
BPW = NBAGS // NW
NCH = BPW // CHB

_c = np.arange(NCOL)
_k = _c % 4
_l = (_c // 16) % L
_h = _c // 64
_loff = np.concatenate([[0], np.cumsum(SPATIAL.prod(1))[:-1]]).astype(np.int64)
CF = np.stack([
    (_k % 2).astype(np.float32),
    (_k // 2).astype(np.float32),
    SPATIAL[_l, 1].astype(np.float32),
    SPATIAL[_l, 0].astype(np.float32),
]).astype(np.float32)
CI = np.stack([
    SPATIAL[_l, 1].astype(np.int64),
    _loff[_l],
    _h,
]).astype(np.int32)
SX = np.zeros((2 * L, NCOL), np.float32)
SY = np.zeros((2 * L, NCOL), np.float32)
SX[2 * _l, _c] = 1.0
SY[2 * _l + 1, _c] = 1.0
GD = (0.25 * (_h[:, None] == _h[None, :])).astype(np.float32)

_HP = lax.Precision.HIGHEST


def _vproj_body(v_ref, w_ref, b_ref, o_ref):
    o_ref[0] = jnp.dot(v_ref[0], w_ref[...], precision=_HP) + b_ref[...]


def _outproj_body(v_ref, w_ref, b_ref, o_ref):
    o_ref[0] = jnp.dot(v_ref[0], w_ref[...], precision=_HP) + b_ref[...]


def _prep_body(q_ref, rp_ref, wx_ref, wy_ref, wa_ref, bx_ref, by_ref, ba_ref,
               sx_ref, sy_ref, gd_ref, cf_ref, ci_ref, idx_ref, w_ref):
    q = q_ref[0]
    ox = jnp.dot(q, wx_ref[...], precision=_HP) + bx_ref[...]
    oy = jnp.dot(q, wy_ref[...], precision=_HP) + by_ref[...]
    ea = jnp.exp(jnp.dot(q, wa_ref[...], precision=_HP) + ba_ref[...])
    aw = ea / jnp.dot(ea, gd_ref[...], precision=_HP)
    rp = rp_ref[0]
    rpx = jnp.dot(rp, sx_ref[...], precision=_HP)
    rpy = jnp.dot(rp, sy_ref[...], precision=_HP)
    kx = cf_ref[0:1]
    ky = cf_ref[1:2]
    wf = cf_ref[2:3]
    hf = cf_ref[3:4]
    x = rpx * wf + ox - 0.5
    y = rpy * hf + oy - 0.5
    x0 = jnp.floor(x)
    fx = x - x0
    y0 = jnp.floor(y)
    fy = y - y0
    xk = x0 + kx
    yk = y0 + ky
    wxw = 1.0 - fx + kx * (2.0 * fx - 1.0)
    wyw = 1.0 - fy + ky * (2.0 * fy - 1.0)
    valid = ((xk >= 0.0) & (xk <= wf - 1.0) & (yk >= 0.0) & (yk <= hf - 1.0))
    xi = jnp.clip(xk, 0.0, wf - 1.0).astype(jnp.int32)
    yi = jnp.clip(yk, 0.0, hf - 1.0).astype(jnp.int32)
    wi = ci_ref[0:1]
    loff = ci_ref[1:2]
    hid = ci_ref[2:3]
    b = pl.program_id(0)
    pos = loff + yi * wi + xi
    idx_ref[0] = (pos + b * LEN_V) * H + hid
    w_ref[0] = aw * wxw * wyw * valid.astype(jnp.float32)


def _vproj(value, W_val, b_val2):
    rb = LEN_V // 4
    return pl.pallas_call(
        _vproj_body,
        grid=(B, 4),
        in_specs=[
            pl.BlockSpec((1, rb, D), lambda b, r: (b, r, 0)),
            pl.BlockSpec((D, D), lambda b, r: (0, 0)),
            pl.BlockSpec((1, D), lambda b, r: (0, 0)),
        ],
        out_specs=pl.BlockSpec((1, rb, D), lambda b, r: (b, r, 0)),
        out_shape=jax.ShapeDtypeStruct((B, LEN_V, D), jnp.float32),
    )(value, W_val, b_val2)


def _outproj(x, W_out, b_out2):
    return pl.pallas_call(
        _outproj_body,
        grid=(B,),
        in_specs=[
            pl.BlockSpec((1, Q, D), lambda b: (b, 0, 0)),
            pl.BlockSpec((D, D), lambda b: (0, 0)),
            pl.BlockSpec((1, D), lambda b: (0, 0)),
        ],
        out_specs=pl.BlockSpec((1, Q, D), lambda b: (b, 0, 0)),
        out_shape=jax.ShapeDtypeStruct((B, Q, D), jnp.float32),
    )(x, W_out, b_out2)


def _prep(query, rp8, Wx, Wy, Wa, bx, by, ba, sx, sy, gd, cf, ci):
    z2 = lambda b, qb: (0, 0)
    return pl.pallas_call(
        _prep_body,
        grid=(B, Q // QB),
        in_specs=[
            pl.BlockSpec((1, QB, D), lambda b, qb: (b, qb, 0)),
            pl.BlockSpec((1, QB, 2 * L), lambda b, qb: (b, qb, 0)),
            pl.BlockSpec((D, NCOL), z2),
            pl.BlockSpec((D, NCOL), z2),
            pl.BlockSpec((D, NCOL), z2),
            pl.BlockSpec((1, NCOL), z2),
            pl.BlockSpec((1, NCOL), z2),
            pl.BlockSpec((1, NCOL), z2),
            pl.BlockSpec((2 * L, NCOL), z2),
            pl.BlockSpec((2 * L, NCOL), z2),
            pl.BlockSpec((NCOL, NCOL), z2),
            pl.BlockSpec((4, NCOL), z2),
            pl.BlockSpec((3, NCOL), z2),
        ],
        out_specs=[
            pl.BlockSpec((1, QB, NCOL), lambda b, qb: (b, qb, 0)),
            pl.BlockSpec((1, QB, NCOL), lambda b, qb: (b, qb, 0)),
        ],
        out_shape=[
            jax.ShapeDtypeStruct((B, Q, NCOL), jnp.int32),
            jax.ShapeDtypeStruct((B, Q, NCOL), jnp.float32),
        ],
    )(query, rp8, Wx, Wy, Wa, bx, by, ba, sx, sy, gd, cf, ci)


def _splat(v, j):
    return lax.gather(
        v, jnp.full((16, 1), j, jnp.int32),
        lax.GatherDimensionNumbers(
            offset_dims=(), collapsed_slice_dims=(0,), start_index_map=(0,)),
        (1,), mode=lax.GatherScatterMode.PROMISE_IN_BOUNDS)


def _sc_body(vtab, idxh, wgth, outh, idx_v, wgt_v, rows_v, out_v,
             isem0, isem1, gsem0, gsem1, osem0, osem1):
    w = lax.axis_index("c") * 16 + lax.axis_index("s")
    isem = (isem0, isem1)
    gsem = (gsem0, gsem1)
    osem = (osem0, osem1)

    def start_iw(ci, p):
        pltpu.make_async_copy(idxh.at[w, ci], idx_v.at[p], isem[p]).start()
        pltpu.make_async_copy(wgth.at[w, ci], wgt_v.at[p], isem[p]).start()

    def wait_iw(p):
        pltpu.make_async_copy(idxh.at[w, 0], idx_v.at[p], isem[p]).wait()
        pltpu.make_async_copy(wgth.at[w, 0], wgt_v.at[p], isem[p]).wait()

    def fire(p):
        for j in range(SS):
            pltpu.make_async_copy(
                vtab.at[idx_v.at[p, j]], rows_v.at[p, j], gsem[p]).start()

    def drain(p):
        for j in range(SS):
            pltpu.make_async_copy(
                vtab.at[idx_v.at[p, j]], rows_v.at[p, j], gsem[p]).wait()

    def compute(p):
        def bag_body(bag, carry):
            jh = bag // 2
            rb = (bag % 2) * 64
            acc0 = jnp.zeros((16,), jnp.float32)
            acc1 = jnp.zeros((16,), jnp.float32)
            for s16 in range(4):
                wblk = wgt_v[p, jh, pl.ds(rb + s16 * 16, 16)]
                for jj in range(16):
                    r = rb + s16 * 16 + jj
                    wv = _splat(wblk, jj)
                    acc0 = acc0 + wv * rows_v[p, jh, r, pl.ds(0, 16)]
                    acc1 = acc1 + wv * rows_v[p, jh, r, pl.ds(16, 16)]
            out_v[p, bag, pl.ds(0, 16)] = acc0
            out_v[p, bag, pl.ds(16, 16)] = acc1
            return carry

        lax.fori_loop(0, CHB, bag_body, 0)

    def step(ci, p):
        @pl.when(ci + 1 < NCH)
        def _():
            wait_iw(1 - p)
            fire(1 - p)

        drain(p)

        @pl.when(ci >= 2)
        def _():
            pltpu.make_async_copy(out_v.at[p], outh.at[w, 0], osem[p]).wait()

        compute(p)
        pltpu.make_async_copy(out_v.at[p], outh.at[w, ci], osem[p]).start()

        @pl.when(ci + 2 < NCH)
        def _():
            start_iw(ci + 2, p)

    start_iw(0, 0)
    wait_iw(0)
    fire(0)
    start_iw(1, 1)

    def outer(i2, carry):
        step(2 * i2, 0)
        step(2 * i2 + 1, 1)
        return carry

    lax.fori_loop(0, NCH // 2, outer, 0)

    pltpu.make_async_copy(out_v.at[0], outh.at[w, 0], osem[0]).wait()
    pltpu.make_async_copy(out_v.at[1], outh.at[w, 0], osem[1]).wait()


@functools.cache
def _sc():
    return pl.kernel(
        _sc_body,
        out_type=jax.ShapeDtypeStruct((NW, NCH, CHB, C), jnp.float32),
        mesh=plsc.VectorSubcoreMesh(core_axis_name="c", subcore_axis_name="s",
                                    num_cores=2, num_subcores=16),
        scratch_types=[
            pltpu.VMEM((2, SS, 128), jnp.int32),
            pltpu.VMEM((2, SS, 128), jnp.float32),
            pltpu.VMEM((2, SS, 128, C), jnp.float32),
            pltpu.VMEM((2, CHB, C), jnp.float32),
            pltpu.SemaphoreType.DMA,
            pltpu.SemaphoreType.DMA,
            pltpu.SemaphoreType.DMA,
            pltpu.SemaphoreType.DMA,
            pltpu.SemaphoreType.DMA,
            pltpu.SemaphoreType.DMA,
        ],
        compiler_params=pltpu.CompilerParams(use_tc_tiling_on_sc=False),
    )


def kernel(query, reference_points, value, value_spatial_shapes,
           W_off, b_off, W_attn, b_attn, W_val, b_val, W_out, b_out):
    Wo = W_off.reshape(D, H, L, P, 2)
    Wx = jnp.broadcast_to(Wo[..., 0][..., None], (D, H, L, P, 4)).reshape(D, NCOL)
    Wy = jnp.broadcast_to(Wo[..., 1][..., None], (D, H, L, P, 4)).reshape(D, NCOL)
    bo = b_off.reshape(H, L, P, 2)
    bx = jnp.broadcast_to(bo[..., 0][..., None], (H, L, P, 4)).reshape(1, NCOL)
    by = jnp.broadcast_to(bo[..., 1][..., None], (H, L, P, 4)).reshape(1, NCOL)
    Wa = jnp.broadcast_to(W_attn.reshape(D, H, L * P)[..., None],
                          (D, H, L * P, 4)).reshape(D, NCOL)
    ba = jnp.broadcast_to(b_attn.reshape(H, L * P)[..., None],
                          (H, L * P, 4)).reshape(1, NCOL)
    rp8 = reference_points.reshape(B, Q, 2 * L)

    v = _vproj(value, W_val, b_val.reshape(1, D))
    idx, wgt = _prep(query, rp8, Wx, Wy, Wa, bx, by, ba,
                     jnp.asarray(SX), jnp.asarray(SY), jnp.asarray(GD),
                     jnp.asarray(CF), jnp.asarray(CI))
    agg = _sc()(v.reshape(B * LEN_V * H, C),
                idx.reshape(NW, NCH, SS, 128),
                wgt.reshape(NW, NCH, SS, 128))
    return _outproj(agg.reshape(B, Q, D), W_out, b_out.reshape(1, D))


# speedup vs baseline: 1.0373x; 1.0373x over previous
import functools

import numpy as np
import jax
import jax.numpy as jnp
from jax import lax
from jax.experimental import pallas as pl
from jax.experimental.pallas import tpu as pltpu
from jax.experimental.pallas import tpu_sc as plsc

D = 256
H = 8
L = 4
P = 4
C = 32
B = 4
Q = 2048
SPATIAL = np.array([[64, 64], [32, 32], [16, 16], [8, 8]], dtype=np.int64)
LEN_V = int(SPATIAL.prod(1).sum())
NCOL = H * L * P * 4

QB = 256
NW = 32
CHB = 16
SS = 8
NBAGS = B * Q * H
BPW = NBAGS // NW
NCH = BPW // CHB

_c = np.arange(NCOL)
_k = _c % 4
_l = (_c // 16) % L
_h = _c // 64
_loff = np.concatenate([[0], np.cumsum(SPATIAL.prod(1))[:-1]]).astype(np.int64)
CF = np.stack([
    (_k % 2).astype(np.float32),
    (_k // 2).astype(np.float32),
    SPATIAL[_l, 1].astype(np.float32),
    SPATIAL[_l, 0].astype(np.float32),
]).astype(np.float32)
CI = np.stack([
    SPATIAL[_l, 1].astype(np.int64),
    _loff[_l],
    _h,
]).astype(np.int32)
SX = np.zeros((2 * L, NCOL), np.float32)
SY = np.zeros((2 * L, NCOL), np.float32)
SX[2 * _l, _c] = 1.0
SY[2 * _l + 1, _c] = 1.0
GD = (0.25 * (_h[:, None] == _h[None, :])).astype(np.float32)

_perm = np.concatenate([np.arange(0, C, 2), np.arange(1, C, 2)])
ROWPERM = np.concatenate([h * C + _perm for h in range(H)])

_HP = lax.Precision.HIGHEST


def _vproj_body(v_ref, w_ref, b_ref, o_ref):
    o_ref[0] = (jnp.dot(v_ref[0], w_ref[...], precision=_HP)
                + b_ref[...]).astype(jnp.bfloat16)


def _outproj_body(v_ref, w_ref, b_ref, o_ref):
    o_ref[0] = jnp.dot(v_ref[0], w_ref[...], precision=_HP) + b_ref[...]


def _prep_body(q_ref, rp_ref, wx_ref, wy_ref, wa_ref, bx_ref, by_ref, ba_ref,
               sx_ref, sy_ref, gd_ref, cf_ref, ci_ref, idx_ref, w_ref):
    q = q_ref[0]
    ox = jnp.dot(q, wx_ref[...], precision=_HP) + bx_ref[...]
    oy = jnp.dot(q, wy_ref[...], precision=_HP) + by_ref[...]
    ea = jnp.exp(jnp.dot(q, wa_ref[...], precision=_HP) + ba_ref[...])
    aw = ea / jnp.dot(ea, gd_ref[...], precision=_HP)
    rp = rp_ref[0]
    rpx = jnp.dot(rp, sx_ref[...], precision=_HP)
    rpy = jnp.dot(rp, sy_ref[...], precision=_HP)
    kx = cf_ref[0:1]
    ky = cf_ref[1:2]
    wf = cf_ref[2:3]
    hf = cf_ref[3:4]
    x = rpx * wf + ox - 0.5
    y = rpy * hf + oy - 0.5
    x0 = jnp.floor(x)
    fx = x - x0
    y0 = jnp.floor(y)
    fy = y - y0
    xk = x0 + kx
    yk = y0 + ky
    wxw = 1.0 - fx + kx * (2.0 * fx - 1.0)
    wyw = 1.0 - fy + ky * (2.0 * fy - 1.0)
    valid = ((xk >= 0.0) & (xk <= wf - 1.0) & (yk >= 0.0) & (yk <= hf - 1.0))
    xi = jnp.clip(xk, 0.0, wf - 1.0).astype(jnp.int32)
    yi = jnp.clip(yk, 0.0, hf - 1.0).astype(jnp.int32)
    wi = ci_ref[0:1]
    loff = ci_ref[1:2]
    hid = ci_ref[2:3]
    b = pl.program_id(0)
    pos = loff + yi * wi + xi
    idx_ref[0] = (pos + b * LEN_V) * H + hid
    w_ref[0] = aw * wxw * wyw * valid.astype(jnp.float32)


def _vproj(value, W_val, b_val2):
    rb = LEN_V // 4
    return pl.pallas_call(
        _vproj_body,
        grid=(B, 4),
        in_specs=[
            pl.BlockSpec((1, rb, D), lambda b, r: (b, r, 0)),
            pl.BlockSpec((D, D), lambda b, r: (0, 0)),
            pl.BlockSpec((1, D), lambda b, r: (0, 0)),
        ],
        out_specs=pl.BlockSpec((1, rb, D), lambda b, r: (b, r, 0)),
        out_shape=jax.ShapeDtypeStruct((B, LEN_V, D), jnp.bfloat16),
    )(value, W_val, b_val2)


def _outproj(x, W_out, b_out2):
    return pl.pallas_call(
        _outproj_body,
        grid=(B,),
        in_specs=[
            pl.BlockSpec((1, Q, D), lambda b: (b, 0, 0)),
            pl.BlockSpec((D, D), lambda b: (0, 0)),
            pl.BlockSpec((1, D), lambda b: (0, 0)),
        ],
        out_specs=pl.BlockSpec((1, Q, D), lambda b: (b, 0, 0)),
        out_shape=jax.ShapeDtypeStruct((B, Q, D), jnp.float32),
    )(x, W_out, b_out2)


def _prep(query, rp8, Wx, Wy, Wa, bx, by, ba, sx, sy, gd, cf, ci):
    z2 = lambda b, qb: (0, 0)
    return pl.pallas_call(
        _prep_body,
        grid=(B, Q // QB),
        in_specs=[
            pl.BlockSpec((1, QB, D), lambda b, qb: (b, qb, 0)),
            pl.BlockSpec((1, QB, 2 * L), lambda b, qb: (b, qb, 0)),
            pl.BlockSpec((D, NCOL), z2),
            pl.BlockSpec((D, NCOL), z2),
            pl.BlockSpec((D, NCOL), z2),
            pl.BlockSpec((1, NCOL), z2),
            pl.BlockSpec((1, NCOL), z2),
            pl.BlockSpec((1, NCOL), z2),
            pl.BlockSpec((2 * L, NCOL), z2),
            pl.BlockSpec((2 * L, NCOL), z2),
            pl.BlockSpec((NCOL, NCOL), z2),
            pl.BlockSpec((4, NCOL), z2),
            pl.BlockSpec((3, NCOL), z2),
        ],
        out_specs=[
            pl.BlockSpec((1, QB, NCOL), lambda b, qb: (b, qb, 0)),
            pl.BlockSpec((1, QB, NCOL), lambda b, qb: (b, qb, 0)),
        ],
        out_shape=[
            jax.ShapeDtypeStruct((B, Q, NCOL), jnp.int32),
            jax.ShapeDtypeStruct((B, Q, NCOL), jnp.float32),
        ],
    )(query, rp8, Wx, Wy, Wa, bx, by, ba, sx, sy, gd, cf, ci)


def _splat(v, j):
    return lax.gather(
        v, jnp.full((16, 1), j, jnp.int32),
        lax.GatherDimensionNumbers(
            offset_dims=(), collapsed_slice_dims=(0,), start_index_map=(0,)),
        (1,), mode=lax.GatherScatterMode.PROMISE_IN_BOUNDS)


def _sc_body(vtab, idxh, wgth, outh, idx_v, wgt_v, rows_v, out_v,
             isem0, isem1, gsem0, gsem1, osem0, osem1):
    w = lax.axis_index("c") * 16 + lax.axis_index("s")
    isem = (isem0, isem1)
    gsem = (gsem0, gsem1)
    osem = (osem0, osem1)

    def start_iw(ci, p):
        pltpu.make_async_copy(idxh.at[w, ci], idx_v.at[p], isem[p]).start()
        pltpu.make_async_copy(wgth.at[w, ci], wgt_v.at[p], isem[p]).start()

    def wait_iw(p):
        pltpu.make_async_copy(idxh.at[w, 0], idx_v.at[p], isem[p]).wait()
        pltpu.make_async_copy(wgth.at[w, 0], wgt_v.at[p], isem[p]).wait()

    def fire(p):
        for j in range(SS):
            pltpu.make_async_copy(
                vtab.at[idx_v.at[p, j]], rows_v.at[p, j], gsem[p]).start()

    def drain(p):
        for j in range(SS):
            pltpu.make_async_copy(
                vtab.at[idx_v.at[p, j]], rows_v.at[p, j], gsem[p]).wait()

    def compute(p):
        def bag_body(bag, carry):
            jh = bag // 2
            rb = (bag % 2) * 64
            acc0 = jnp.zeros((16,), jnp.float32)
            acc1 = jnp.zeros((16,), jnp.float32)
            for s16 in range(4):
                wblk = wgt_v[p, jh, pl.ds(rb + s16 * 16, 16)]
                for jj in range(16):
                    r = rb + s16 * 16 + jj
                    wv = _splat(wblk, jj)
                    ra, rc = plsc.unpack(rows_v[p, jh, r, :],
                                         format=plsc.PackFormat.INTERLEAVED)
                    acc0 = acc0 + wv * ra
                    acc1 = acc1 + wv * rc
            out_v[p, bag, pl.ds(0, 16)] = acc0
            out_v[p, bag, pl.ds(16, 16)] = acc1
            return carry

        lax.fori_loop(0, CHB, bag_body, 0)

    def step(ci, p):
        @pl.when(ci + 1 < NCH)
        def _():
            wait_iw(1 - p)
            fire(1 - p)

        drain(p)

        @pl.when(ci >= 2)
        def _():
            pltpu.make_async_copy(out_v.at[p], outh.at[w, 0], osem[p]).wait()

        compute(p)
        pltpu.make_async_copy(out_v.at[p], outh.at[w, ci], osem[p]).start()

        @pl.when(ci + 2 < NCH)
        def _():
            start_iw(ci + 2, p)

    start_iw(0, 0)
    wait_iw(0)
    fire(0)
    start_iw(1, 1)

    def outer(i2, carry):
        step(2 * i2, 0)
        step(2 * i2 + 1, 1)
        return carry

    lax.fori_loop(0, NCH // 2, outer, 0)

    pltpu.make_async_copy(out_v.at[0], outh.at[w, 0], osem[0]).wait()
    pltpu.make_async_copy(out_v.at[1], outh.at[w, 0], osem[1]).wait()


@functools.cache
def _sc():
    return pl.kernel(
        _sc_body,
        out_type=jax.ShapeDtypeStruct((NW, NCH, CHB, C), jnp.float32),
        mesh=plsc.VectorSubcoreMesh(core_axis_name="c", subcore_axis_name="s",
                                    num_cores=2, num_subcores=16),
        scratch_types=[
            pltpu.VMEM((2, SS, 128), jnp.int32),
            pltpu.VMEM((2, SS, 128), jnp.float32),
            pltpu.VMEM((2, SS, 128, C), jnp.bfloat16),
            pltpu.VMEM((2, CHB, C), jnp.float32),
            pltpu.SemaphoreType.DMA,
            pltpu.SemaphoreType.DMA,
            pltpu.SemaphoreType.DMA,
            pltpu.SemaphoreType.DMA,
            pltpu.SemaphoreType.DMA,
            pltpu.SemaphoreType.DMA,
        ],
        compiler_params=pltpu.CompilerParams(use_tc_tiling_on_sc=False,
                                             needs_layout_passes=False),
    )


def kernel(query, reference_points, value, value_spatial_shapes,
           W_off, b_off, W_attn, b_attn, W_val, b_val, W_out, b_out):
    Wo = W_off.reshape(D, H, L, P, 2)
    Wx = jnp.broadcast_to(Wo[..., 0][..., None], (D, H, L, P, 4)).reshape(D, NCOL)
    Wy = jnp.broadcast_to(Wo[..., 1][..., None], (D, H, L, P, 4)).reshape(D, NCOL)
    bo = b_off.reshape(H, L, P, 2)
    bx = jnp.broadcast_to(bo[..., 0][..., None], (H, L, P, 4)).reshape(1, NCOL)
    by = jnp.broadcast_to(bo[..., 1][..., None], (H, L, P, 4)).reshape(1, NCOL)
    Wa = jnp.broadcast_to(W_attn.reshape(D, H, L * P)[..., None],
                          (D, H, L * P, 4)).reshape(D, NCOL)
    ba = jnp.broadcast_to(b_attn.reshape(H, L * P)[..., None],
                          (H, L * P, 4)).reshape(1, NCOL)
    rp8 = reference_points.reshape(B, Q, 2 * L)

    v = _vproj(value, W_val, b_val.reshape(1, D))
    idx, wgt = _prep(query, rp8, Wx, Wy, Wa, bx, by, ba,
                     jnp.asarray(SX), jnp.asarray(SY), jnp.asarray(GD),
                     jnp.asarray(CF), jnp.asarray(CI))
    agg = _sc()(v.reshape(B * LEN_V * H, C),
                idx.reshape(NW, NCH, SS, 128),
                wgt.reshape(NW, NCH, SS, 128))
    return _outproj(agg.reshape(B, Q, D), W_out[jnp.asarray(ROWPERM)],
                    b_out.reshape(1, D))


# speedup vs baseline: 1.2262x; 1.1821x over previous
import functools

import numpy as np
import jax
import jax.numpy as jnp
from jax import lax
from jax.experimental import pallas as pl
from jax.experimental.pallas import tpu as pltpu
from jax.experimental.pallas import tpu_sc as plsc

D = 256
H = 8
L = 4
P = 4
C = 32
B = 4
Q = 2048
SPATIAL = np.array([[64, 64], [32, 32], [16, 16], [8, 8]], dtype=np.int64)
LEN_V = int(SPATIAL.prod(1).sum())
NCOL = H * L * P * 4

QB = 256
NW = 32
CHB = 16
SS = 8
NBAGS = B * Q * H
BPW = NBAGS // NW
NCH = BPW // CHB

_c = np.arange(NCOL)
_k = _c % 4
_l = (_c // 16) % L
_h = _c // 64
_loff = np.concatenate([[0], np.cumsum(SPATIAL.prod(1))[:-1]]).astype(np.int64)
CF = np.stack([
    (_k % 2).astype(np.float32),
    (_k // 2).astype(np.float32),
    SPATIAL[_l, 1].astype(np.float32),
    SPATIAL[_l, 0].astype(np.float32),
]).astype(np.float32)
CI = np.stack([
    SPATIAL[_l, 1].astype(np.int64),
    _loff[_l],
    _h,
]).astype(np.int32)
SX = np.zeros((2 * L, NCOL), np.float32)
SY = np.zeros((2 * L, NCOL), np.float32)
SX[2 * _l, _c] = 1.0
SY[2 * _l + 1, _c] = 1.0
GD = (0.25 * (_h[:, None] == _h[None, :])).astype(np.float32)

_perm = np.concatenate([np.arange(0, C, 2), np.arange(1, C, 2)])
ROWPERM = np.concatenate([h * C + _perm for h in range(H)])


def _vproj_body(v_ref, w_ref, b_ref, o_ref):
    o_ref[0] = (jnp.dot(v_ref[0], w_ref[...])
                + b_ref[...]).astype(jnp.bfloat16)


def _outproj_body(v_ref, w_ref, b_ref, o_ref):
    o_ref[0] = jnp.dot(v_ref[0], w_ref[...]) + b_ref[...]


def _prep_body(q_ref, rp_ref, wx_ref, wy_ref, wa_ref, bx_ref, by_ref, ba_ref,
               sx_ref, sy_ref, gd_ref, cf_ref, ci_ref, idx_ref, w_ref):
    q = q_ref[0]
    ox = jnp.dot(q, wx_ref[...]) + bx_ref[...]
    oy = jnp.dot(q, wy_ref[...]) + by_ref[...]
    ea = jnp.exp(jnp.dot(q, wa_ref[...]) + ba_ref[...])
    aw = ea / jnp.dot(ea, gd_ref[...])
    rp = rp_ref[0]
    rpx = jnp.dot(rp, sx_ref[...])
    rpy = jnp.dot(rp, sy_ref[...])
    kx = cf_ref[0:1]
    ky = cf_ref[1:2]
    wf = cf_ref[2:3]
    hf = cf_ref[3:4]
    x = rpx * wf + ox - 0.5
    y = rpy * hf + oy - 0.5
    x0 = jnp.floor(x)
    fx = x - x0
    y0 = jnp.floor(y)
    fy = y - y0
    xk = x0 + kx
    yk = y0 + ky
    wxw = 1.0 - fx + kx * (2.0 * fx - 1.0)
    wyw = 1.0 - fy + ky * (2.0 * fy - 1.0)
    valid = ((xk >= 0.0) & (xk <= wf - 1.0) & (yk >= 0.0) & (yk <= hf - 1.0))
    xi = jnp.clip(xk, 0.0, wf - 1.0).astype(jnp.int32)
    yi = jnp.clip(yk, 0.0, hf - 1.0).astype(jnp.int32)
    wi = ci_ref[0:1]
    loff = ci_ref[1:2]
    hid = ci_ref[2:3]
    b = pl.program_id(0)
    pos = loff + yi * wi + xi
    idx_ref[0] = (pos + b * LEN_V) * H + hid
    w_ref[0] = aw * wxw * wyw * valid.astype(jnp.float32)


def _vproj(value, W_val, b_val2):
    rb = LEN_V // 4
    return pl.pallas_call(
        _vproj_body,
        grid=(B, 4),
        in_specs=[
            pl.BlockSpec((1, rb, D), lambda b, r: (b, r, 0)),
            pl.BlockSpec((D, D), lambda b, r: (0, 0)),
            pl.BlockSpec((1, D), lambda b, r: (0, 0)),
        ],
        out_specs=pl.BlockSpec((1, rb, D), lambda b, r: (b, r, 0)),
        out_shape=jax.ShapeDtypeStruct((B, LEN_V, D), jnp.bfloat16),
    )(value, W_val, b_val2)


def _outproj(x, W_out, b_out2):
    return pl.pallas_call(
        _outproj_body,
        grid=(B,),
        in_specs=[
            pl.BlockSpec((1, Q, D), lambda b: (b, 0, 0)),
            pl.BlockSpec((D, D), lambda b: (0, 0)),
            pl.BlockSpec((1, D), lambda b: (0, 0)),
        ],
        out_specs=pl.BlockSpec((1, Q, D), lambda b: (b, 0, 0)),
        out_shape=jax.ShapeDtypeStruct((B, Q, D), jnp.float32),
    )(x, W_out, b_out2)


def _prep(query, rp8, Wx, Wy, Wa, bx, by, ba, sx, sy, gd, cf, ci):
    z2 = lambda b, qb: (0, 0)
    return pl.pallas_call(
        _prep_body,
        grid=(B, Q // QB),
        in_specs=[
            pl.BlockSpec((1, QB, D), lambda b, qb: (b, qb, 0)),
            pl.BlockSpec((1, QB, 2 * L), lambda b, qb: (b, qb, 0)),
            pl.BlockSpec((D, NCOL), z2),
            pl.BlockSpec((D, NCOL), z2),
            pl.BlockSpec((D, NCOL), z2),
            pl.BlockSpec((1, NCOL), z2),
            pl.BlockSpec((1, NCOL), z2),
            pl.BlockSpec((1, NCOL), z2),
            pl.BlockSpec((2 * L, NCOL), z2),
            pl.BlockSpec((2 * L, NCOL), z2),
            pl.BlockSpec((NCOL, NCOL), z2),
            pl.BlockSpec((4, NCOL), z2),
            pl.BlockSpec((3, NCOL), z2),
        ],
        out_specs=[
            pl.BlockSpec((1, QB, NCOL), lambda b, qb: (b, qb, 0)),
            pl.BlockSpec((1, QB, NCOL), lambda b, qb: (b, qb, 0)),
        ],
        out_shape=[
            jax.ShapeDtypeStruct((B, Q, NCOL), jnp.int32),
            jax.ShapeDtypeStruct((B, Q, NCOL), jnp.float32),
        ],
    )(query, rp8, Wx, Wy, Wa, bx, by, ba, sx, sy, gd, cf, ci)


def _splat(v, j):
    return lax.gather(
        v, jnp.full((16, 1), j, jnp.int32),
        lax.GatherDimensionNumbers(
            offset_dims=(), collapsed_slice_dims=(0,), start_index_map=(0,)),
        (1,), mode=lax.GatherScatterMode.PROMISE_IN_BOUNDS)


def _sc_body(vtab, idxh, wgth, outh, idx_v, wgt_v, rows_v, out_v,
             isem0, isem1, gsem0, gsem1, osem0, osem1):
    w = lax.axis_index("c") * 16 + lax.axis_index("s")
    isem = (isem0, isem1)
    gsem = (gsem0, gsem1)
    osem = (osem0, osem1)

    def start_iw(ci, p):
        pltpu.make_async_copy(idxh.at[w, ci], idx_v.at[p], isem[p]).start()
        pltpu.make_async_copy(wgth.at[w, ci], wgt_v.at[p], isem[p]).start()

    def wait_iw(p):
        pltpu.make_async_copy(idxh.at[w, 0], idx_v.at[p], isem[p]).wait()
        pltpu.make_async_copy(wgth.at[w, 0], wgt_v.at[p], isem[p]).wait()

    def fire(p):
        for j in range(SS):
            pltpu.make_async_copy(
                vtab.at[idx_v.at[p, j]], rows_v.at[p, j], gsem[p]).start()

    def drain(p):
        for j in range(SS):
            pltpu.make_async_copy(
                vtab.at[idx_v.at[p, j]], rows_v.at[p, j], gsem[p]).wait()

    def compute(p):
        def bag_body(bag, carry):
            jh = bag // 2
            rb = (bag % 2) * 64
            acc0 = jnp.zeros((16,), jnp.float32)
            acc1 = jnp.zeros((16,), jnp.float32)
            for s16 in range(4):
                wblk = wgt_v[p, jh, pl.ds(rb + s16 * 16, 16)]
                for jj in range(16):
                    r = rb + s16 * 16 + jj
                    wv = _splat(wblk, jj)
                    ri = plsc.bitcast(rows_v[p, jh, r, :], jnp.int32)
                    ra = plsc.bitcast(lax.shift_left(ri, 16), jnp.float32)
                    rc = plsc.bitcast(
                        jnp.bitwise_and(ri, jnp.int32(-65536)), jnp.float32)
                    acc0 = acc0 + wv * ra
                    acc1 = acc1 + wv * rc
            out_v[p, bag, pl.ds(0, 16)] = acc0
            out_v[p, bag, pl.ds(16, 16)] = acc1
            return carry

        lax.fori_loop(0, CHB, bag_body, 0)

    def step(ci, p):
        @pl.when(ci + 1 < NCH)
        def _():
            wait_iw(1 - p)
            fire(1 - p)

        drain(p)

        @pl.when(ci >= 2)
        def _():
            pltpu.make_async_copy(out_v.at[p], outh.at[w, 0], osem[p]).wait()

        compute(p)
        pltpu.make_async_copy(out_v.at[p], outh.at[w, ci], osem[p]).start()

        @pl.when(ci + 2 < NCH)
        def _():
            start_iw(ci + 2, p)

    start_iw(0, 0)
    wait_iw(0)
    fire(0)
    start_iw(1, 1)

    def outer(i2, carry):
        step(2 * i2, 0)
        step(2 * i2 + 1, 1)
        return carry

    lax.fori_loop(0, NCH // 2, outer, 0)

    pltpu.make_async_copy(out_v.at[0], outh.at[w, 0], osem[0]).wait()
    pltpu.make_async_copy(out_v.at[1], outh.at[w, 0], osem[1]).wait()


@functools.cache
def _sc():
    return pl.kernel(
        _sc_body,
        out_type=jax.ShapeDtypeStruct((NW, NCH, CHB, C), jnp.float32),
        mesh=plsc.VectorSubcoreMesh(core_axis_name="c", subcore_axis_name="s",
                                    num_cores=2, num_subcores=16),
        scratch_types=[
            pltpu.VMEM((2, SS, 128), jnp.int32),
            pltpu.VMEM((2, SS, 128), jnp.float32),
            pltpu.VMEM((2, SS, 128, C), jnp.bfloat16),
            pltpu.VMEM((2, CHB, C), jnp.float32),
            pltpu.SemaphoreType.DMA,
            pltpu.SemaphoreType.DMA,
            pltpu.SemaphoreType.DMA,
            pltpu.SemaphoreType.DMA,
            pltpu.SemaphoreType.DMA,
            pltpu.SemaphoreType.DMA,
        ],
        compiler_params=pltpu.CompilerParams(use_tc_tiling_on_sc=False,
                                             needs_layout_passes=False),
    )


def kernel(query, reference_points, value, value_spatial_shapes,
           W_off, b_off, W_attn, b_attn, W_val, b_val, W_out, b_out):
    Wo = W_off.reshape(D, H, L, P, 2)
    Wx = jnp.broadcast_to(Wo[..., 0][..., None], (D, H, L, P, 4)).reshape(D, NCOL)
    Wy = jnp.broadcast_to(Wo[..., 1][..., None], (D, H, L, P, 4)).reshape(D, NCOL)
    bo = b_off.reshape(H, L, P, 2)
    bx = jnp.broadcast_to(bo[..., 0][..., None], (H, L, P, 4)).reshape(1, NCOL)
    by = jnp.broadcast_to(bo[..., 1][..., None], (H, L, P, 4)).reshape(1, NCOL)
    Wa = jnp.broadcast_to(W_attn.reshape(D, H, L * P)[..., None],
                          (D, H, L * P, 4)).reshape(D, NCOL)
    ba = jnp.broadcast_to(b_attn.reshape(H, L * P)[..., None],
                          (H, L * P, 4)).reshape(1, NCOL)
    rp8 = reference_points.reshape(B, Q, 2 * L)

    v = _vproj(value, W_val, b_val.reshape(1, D))
    idx, wgt = _prep(query, rp8, Wx, Wy, Wa, bx, by, ba,
                     jnp.asarray(SX), jnp.asarray(SY), jnp.asarray(GD),
                     jnp.asarray(CF), jnp.asarray(CI))
    agg = _sc()(v.reshape(B * LEN_V * H, C),
                idx.reshape(NW, NCH, SS, 128),
                wgt.reshape(NW, NCH, SS, 128))
    return _outproj(agg.reshape(B, Q, D), W_out[jnp.asarray(ROWPERM)],
                    b_out.reshape(1, D))
